# baseline (speedup 1.0000x reference)
import functools

import jax
import jax.numpy as jnp
from jax import lax
from jax.experimental import pallas as pl
from jax.experimental.pallas import tpu as pltpu
from jax.experimental.pallas import tpu_sc as plsc

_NC, _NS = 2, 16
_NW = _NC * _NS
_B = 2
_HW = 256
_D = 256
_N = 512
_CPW = _D // _NW
_L = 16


def _tc_scores_body(x_ref, w_ref, amin_ref):
    w = w_ref[...]
    e2 = jnp.sum(w * w, axis=0, keepdims=True)
    for b in range(_B):
        xb = x_ref[b]
        s = lax.dot_general(
            xb, w, (((0,), (0,)), ((), ())),
            preferred_element_type=jnp.float32,
            precision=lax.Precision.HIGHEST)
        score = e2 - 2.0 * s
        mn = jnp.min(score, axis=1, keepdims=True)
        ii = lax.broadcasted_iota(jnp.int32, score.shape, 1)
        amin_ref[b, :] = jnp.min(jnp.where(score <= mn, ii, _N), axis=1)


_tc_scores = pl.pallas_call(
    _tc_scores_body,
    out_shape=jax.ShapeDtypeStruct((_B, _HW), jnp.int32),
)


def _sc_lookup_body(w_hbm, idx_hbm, out_hbm, w_v, idx_v, out_v):
    wid = lax.axis_index("s") * _NC + lax.axis_index("c")
    cbase = wid * _CPW
    pltpu.sync_copy(w_hbm.at[pl.ds(cbase, _CPW)], w_v)
    pltpu.sync_copy(idx_hbm, idx_v)
    for b in range(_B):
        for c in range(_CPW):
            c_vec = jnp.full((_L,), c, jnp.int32)
            for k in range(_HW // _L):
                cols = idx_v[b, pl.ds(k * _L, _L)]
                out_v[b, c, pl.ds(k * _L, _L)] = plsc.load_gather(
                    w_v, [c_vec, cols])
    pltpu.sync_copy(out_v, out_hbm.at[:, pl.ds(cbase, _CPW)])


@functools.cache
def _sc_lookup():
    return functools.partial(
        pl.kernel,
        out_type=jax.ShapeDtypeStruct((_B, _D, _HW), jnp.float32),
        mesh=plsc.VectorSubcoreMesh(
            core_axis_name="c", subcore_axis_name="s",
            num_cores=_NC, num_subcores=_NS),
        compiler_params=pltpu.CompilerParams(
            use_tc_tiling_on_sc=False, needs_layout_passes=False),
        scratch_types=[
            pltpu.VMEM((_CPW, _N), jnp.float32),
            pltpu.VMEM((_B, _HW), jnp.int32),
            pltpu.VMEM((_B, _CPW, _HW), jnp.float32),
        ],
    )(_sc_lookup_body)


def kernel(x, weight):
    B, D, H, W = x.shape
    x3 = x.reshape(B, D, H * W)
    amin2 = _tc_scores(x3, weight)
    out = _sc_lookup()(weight, amin2)
    return out.reshape(B, D, H, W), amin2.reshape(B, H, W)


# speedup vs baseline: 2.9425x; 2.9425x over previous
import jax
import jax.numpy as jnp
from jax import lax
from jax.experimental import pallas as pl

_B, _HW, _D, _N = 2, 256, 256, 512


def _tc_all_body(x_ref, w_ref, amin_ref, res_ref):
    w = w_ref[...]
    e2 = jnp.sum(w * w, axis=0, keepdims=True)
    for b in range(_B):
        xb = x_ref[b]
        s = lax.dot_general(xb, w, (((0,), (0,)), ((), ())),
                            preferred_element_type=jnp.float32,
                            precision=lax.Precision.HIGHEST)
        score = e2 - 2.0 * s
        mn = jnp.min(score, axis=1, keepdims=True)
        ii = lax.broadcasted_iota(jnp.int32, score.shape, 1)
        am = jnp.min(jnp.where(score <= mn, ii, _N), axis=1)
        amin_ref[b, :] = am
        ohT = (lax.broadcasted_iota(jnp.int32, (_N, _HW), 0) == am[None, :]
               ).astype(jnp.float32)
        res_ref[b] = lax.dot_general(w, ohT, (((1,), (0,)), ((), ())),
                                     preferred_element_type=jnp.float32,
                                     precision=lax.Precision.HIGHEST)


_tc_all = pl.pallas_call(
    _tc_all_body,
    out_shape=(jax.ShapeDtypeStruct((_B, _HW), jnp.int32),
               jax.ShapeDtypeStruct((_B, _D, _HW), jnp.float32)),
)


def kernel(x, weight):
    B, D, H, W = x.shape
    amin2, res = _tc_all(x.reshape(B, D, H * W), weight)
    return res.reshape(B, D, H, W), amin2.reshape(B, H, W)
